# baseline (speedup 1.0000x reference)
import functools

import jax
import jax.numpy as jnp
from jax import lax
from jax.experimental import pallas as pl
from jax.experimental.pallas import tpu as pltpu
from jax.experimental.pallas import tpu_sc as plsc

NEG = -3.0e38
BIG_ID = 2**31 - 1
CHUNK = 8192
NKEYS = 1_000_000
NCHUNKS = 123
PAD_KEYS = NCHUNKS * CHUNK - NKEYS
TOPG = 4
TOPE = 16
K = 100


def _chunk_kernel(q_ref, k_ref, os_ref, oi_ref, s_ref):
    c = pl.program_id(0)
    q = q_ref[...]
    kk = k_ref[...]
    s = lax.dot_general(q, kk, (((1,), (1,)), ((), ())),
                        preferred_element_type=jnp.float32)
    col = lax.broadcasted_iota(jnp.int32, (64, CHUNK), 1)
    gid = col + c * CHUNK
    s_ref[...] = jnp.where(gid < NKEYS, s, NEG)

    qb = 16
    lane = lax.broadcasted_iota(jnp.int32, (qb, 128), 1) + c * CHUNK
    all_s, all_i = [], []
    for qg in range(64 // qb):
        rs = [jnp.full((qb, 128), NEG, jnp.float32) for _ in range(TOPG)]
        ri = [jnp.full((qb, 128), BIG_ID, jnp.int32) for _ in range(TOPG)]
        for g in range(CHUNK // 128):
            v = s_ref[qg * qb:(qg + 1) * qb, g * 128:(g + 1) * 128]
            vi = lane + (g * 128)
            for t in range(TOPG):
                swap = v > rs[t]
                rs[t], v = jnp.where(swap, v, rs[t]), jnp.where(swap, rs[t], v)
                ri[t], vi = (jnp.where(swap, vi, ri[t]),
                             jnp.where(swap, ri[t], vi))

        cand = jnp.concatenate([r.reshape(qb, 1, 128) for r in rs],
                               axis=1).reshape(qb, TOPG * 128)
        candi = jnp.concatenate([r.reshape(qb, 1, 128) for r in ri],
                                axis=1).reshape(qb, TOPG * 128)
        outs, outis = [], []
        for _ in range(TOPE):
            m = jnp.max(cand, axis=1, keepdims=True)
            idc = jnp.where(cand == m, candi, BIG_ID)
            mi = jnp.min(idc, axis=1, keepdims=True)
            outs.append(m)
            outis.append(mi)
            cand = jnp.where((cand == m) & (candi == mi), NEG, cand)
        all_s.append(jnp.concatenate(outs, axis=1))
        all_i.append(jnp.concatenate(outis, axis=1))
    os_ref[0] = jnp.concatenate(all_s, axis=0)
    oi_ref[0] = jnp.concatenate(all_i, axis=0)


def _select_kernel(cs_ref, ci_ref, ts_ref, ti_ref):
    cand = cs_ref[...]
    candi = ci_ref[...]
    outs, outis = [], []
    for _ in range(K):
        m = jnp.max(cand, axis=1, keepdims=True)
        idc = jnp.where(cand == m, candi, BIG_ID)
        mi = jnp.min(idc, axis=1, keepdims=True)
        outs.append(m)
        outis.append(mi)
        cand = jnp.where((cand == m) & (candi == mi), NEG, cand)
    ts_ref[...] = jnp.concatenate(outs, axis=1)
    ti_ref[...] = jnp.concatenate(outis, axis=1)


def _sc_gather(table1d, idx3):
    nw, r, _ = idx3.shape
    mesh = plsc.VectorSubcoreMesh(core_axis_name="c", subcore_axis_name="s")

    @functools.partial(
        pl.kernel,
        mesh=mesh,
        out_type=jax.ShapeDtypeStruct((nw, r, 128), jnp.float32),
        scratch_types=[
            pltpu.VMEM((r, 128), jnp.int32),
            pltpu.VMEM((r, 128), jnp.float32),
            pltpu.SemaphoreType.DMA,
        ],
    )
    def k(table_hbm, idx_hbm, out_hbm, idx_v, rows_v, sem):
        wid = lax.axis_index("s") * 2 + lax.axis_index("c")
        pltpu.sync_copy(idx_hbm.at[wid], idx_v)
        copies = [
            pltpu.async_copy(table_hbm.at[idx_v.at[j]], rows_v.at[j], sem)
            for j in range(r)
        ]
        for cp in copies:
            cp.wait()
        pltpu.sync_copy(rows_v, out_hbm.at[wid])

    return k(table1d, idx3)


def kernel(queries, keys, k):
    cand_s, cand_i = pl.pallas_call(
        _chunk_kernel,
        grid=(NCHUNKS,),
        in_specs=[
            pl.BlockSpec((64, 16), lambda c: (0, 0)),
            pl.BlockSpec((CHUNK, 16), lambda c: (c, 0)),
        ],
        out_specs=[
            pl.BlockSpec((1, 64, TOPE), lambda c: (c, 0, 0)),
            pl.BlockSpec((1, 64, TOPE), lambda c: (c, 0, 0)),
        ],
        out_shape=[
            jax.ShapeDtypeStruct((NCHUNKS, 64, TOPE), jnp.float32),
            jax.ShapeDtypeStruct((NCHUNKS, 64, TOPE), jnp.int32),
        ],
        scratch_shapes=[pltpu.VMEM((64, CHUNK), jnp.float32)],
    )(queries, keys)

    cs = cand_s.transpose(1, 0, 2).reshape(64, NCHUNKS * TOPE)
    ci = cand_i.transpose(1, 0, 2).reshape(64, NCHUNKS * TOPE)

    topk_scores, topk_ids = pl.pallas_call(
        _select_kernel,
        out_shape=[
            jax.ShapeDtypeStruct((64, K), jnp.float32),
            jax.ShapeDtypeStruct((64, K), jnp.int32),
        ],
    )(cs, ci)

    idx16 = (topk_ids.reshape(64 * K, 1)
             + jnp.arange(16, dtype=jnp.int32)[None, :] * NKEYS
             ).reshape(32, 25, 128)
    emb = _sc_gather(keys.T.reshape(NKEYS * 16), idx16)
    return (topk_scores, topk_ids, emb.reshape(64, K, 16))


# speedup vs baseline: 2.1459x; 2.1459x over previous
import functools

import jax
import jax.numpy as jnp
from jax import lax
from jax.experimental import pallas as pl
from jax.experimental.pallas import tpu as pltpu
from jax.experimental.pallas import tpu_sc as plsc

NEG = -3.0e38
BIG_ID = 2**31 - 1
CHUNK = 8192
NKEYS = 1_000_000
NCHUNKS = 123
PAD_KEYS = NCHUNKS * CHUNK - NKEYS
TOPG = 4
TOPE = 16
K = 100


def _chunk_kernel(q_ref, k_ref, os_ref, oi_ref, kt_ref, s_ref):
    c = pl.program_id(0)
    q = q_ref[...]
    kk = k_ref[...]
    kkt = kk.T
    for j in range(CHUNK // 128):
        kt_ref[j] = kkt[:, j * 128:(j + 1) * 128]
    s = lax.dot_general(q, kk, (((1,), (1,)), ((), ())),
                        preferred_element_type=jnp.float32)
    col = lax.broadcasted_iota(jnp.int32, (64, CHUNK), 1)
    gid = col + c * CHUNK
    s_ref[...] = jnp.where(gid < NKEYS, s, NEG)

    qb = 16
    lane = lax.broadcasted_iota(jnp.int32, (qb, 128), 1) + c * CHUNK
    all_s, all_i = [], []
    for qg in range(64 // qb):
        rs = [jnp.full((qb, 128), NEG, jnp.float32) for _ in range(TOPG)]
        ri = [jnp.full((qb, 128), BIG_ID, jnp.int32) for _ in range(TOPG)]
        for g in range(CHUNK // 128):
            v = s_ref[qg * qb:(qg + 1) * qb, g * 128:(g + 1) * 128]
            vi = lane + (g * 128)
            for t in range(TOPG):
                swap = v > rs[t]
                rs[t], v = jnp.where(swap, v, rs[t]), jnp.where(swap, rs[t], v)
                ri[t], vi = (jnp.where(swap, vi, ri[t]),
                             jnp.where(swap, ri[t], vi))

        cand = jnp.concatenate([r.reshape(qb, 1, 128) for r in rs],
                               axis=1).reshape(qb, TOPG * 128)
        candi = jnp.concatenate([r.reshape(qb, 1, 128) for r in ri],
                                axis=1).reshape(qb, TOPG * 128)
        outs, outis = [], []
        for _ in range(TOPE):
            m = jnp.max(cand, axis=1, keepdims=True)
            idc = jnp.where(cand == m, candi, BIG_ID)
            mi = jnp.min(idc, axis=1, keepdims=True)
            outs.append(m)
            outis.append(mi)
            cand = jnp.where((cand == m) & (candi == mi), NEG, cand)
        all_s.append(jnp.concatenate(outs, axis=1))
        all_i.append(jnp.concatenate(outis, axis=1))
    os_ref[0] = jnp.concatenate(all_s, axis=0)
    oi_ref[0] = jnp.concatenate(all_i, axis=0)


def _select_kernel(cs_ref, ci_ref, ts_ref, ti_ref):
    cand = cs_ref[...]
    candi = ci_ref[...]
    outs, outis = [], []
    for _ in range(K):
        m = jnp.max(cand, axis=1, keepdims=True)
        idc = jnp.where(cand == m, candi, BIG_ID)
        mi = jnp.min(idc, axis=1, keepdims=True)
        outs.append(m)
        outis.append(mi)
        cand = jnp.where((cand == m) & (candi == mi), NEG, cand)
    ts_ref[...] = jnp.concatenate(outs, axis=1)
    ti_ref[...] = jnp.concatenate(outis, axis=1)


def _sc_gather(table1d, idx3):
    nw, r, _ = idx3.shape
    mesh = plsc.VectorSubcoreMesh(core_axis_name="c", subcore_axis_name="s")

    @functools.partial(
        pl.kernel,
        mesh=mesh,
        out_type=jax.ShapeDtypeStruct((nw, r, 128), jnp.float32),
        scratch_types=[
            pltpu.VMEM((r, 128), jnp.int32),
            pltpu.VMEM((r, 128), jnp.float32),
            pltpu.SemaphoreType.DMA,
        ],
    )
    def k(table_hbm, idx_hbm, out_hbm, idx_v, rows_v, sem):
        wid = lax.axis_index("s") * 2 + lax.axis_index("c")
        pltpu.sync_copy(idx_hbm.at[wid], idx_v)
        copies = [
            pltpu.async_copy(table_hbm.at[idx_v.at[j]], rows_v.at[j], sem)
            for j in range(r)
        ]
        for cp in copies:
            cp.wait()
        pltpu.sync_copy(rows_v, out_hbm.at[wid])

    return k(table1d, idx3)


def kernel(queries, keys, k):
    cand_s, cand_i, ktr = pl.pallas_call(
        _chunk_kernel,
        grid=(NCHUNKS,),
        in_specs=[
            pl.BlockSpec((64, 16), lambda c: (0, 0)),
            pl.BlockSpec((CHUNK, 16), lambda c: (c, 0)),
        ],
        out_specs=[
            pl.BlockSpec((1, 64, TOPE), lambda c: (c, 0, 0)),
            pl.BlockSpec((1, 64, TOPE), lambda c: (c, 0, 0)),
            pl.BlockSpec((CHUNK // 128, 16, 128), lambda c: (c, 0, 0)),
        ],
        out_shape=[
            jax.ShapeDtypeStruct((NCHUNKS, 64, TOPE), jnp.float32),
            jax.ShapeDtypeStruct((NCHUNKS, 64, TOPE), jnp.int32),
            jax.ShapeDtypeStruct((NCHUNKS * CHUNK // 128, 16, 128),
                                 jnp.float32),
        ],
        scratch_shapes=[pltpu.VMEM((64, CHUNK), jnp.float32)],
    )(queries, keys)

    cs = cand_s.transpose(1, 0, 2).reshape(64, NCHUNKS * TOPE)
    ci = cand_i.transpose(1, 0, 2).reshape(64, NCHUNKS * TOPE)

    topk_scores, topk_ids = pl.pallas_call(
        _select_kernel,
        out_shape=[
            jax.ShapeDtypeStruct((64, K), jnp.float32),
            jax.ShapeDtypeStruct((64, K), jnp.int32),
        ],
    )(cs, ci)

    base = (topk_ids // 128) * 2048 + (topk_ids % 128)
    idx16 = (base.reshape(64 * K, 1)
             + jnp.arange(16, dtype=jnp.int32)[None, :] * 128
             ).reshape(32, 25, 128)
    emb = _sc_gather(ktr.reshape(NCHUNKS * CHUNK * 16), idx16)
    return (topk_scores, topk_ids, emb.reshape(64, K, 16))


# speedup vs baseline: 2.3888x; 1.1132x over previous
import functools

import jax
import jax.numpy as jnp
from jax import lax
from jax.experimental import pallas as pl
from jax.experimental.pallas import tpu as pltpu
from jax.experimental.pallas import tpu_sc as plsc

NEG = -3.0e38
BIG_ID = 2**31 - 1
CHUNK = 8192
NKEYS = 1_000_000
NCHUNKS = 123
PAD_KEYS = NCHUNKS * CHUNK - NKEYS
TOPG = 4
TOPE = 12
K = 100


def _chunk_kernel(q_ref, k_ref, os_ref, oi_ref, kt_ref, s_ref):
    c = pl.program_id(0)
    q = q_ref[...]
    kk = k_ref[...]
    kkt = kk.T
    for j in range(CHUNK // 128):
        kt_ref[j] = kkt[:, j * 128:(j + 1) * 128]
    s = lax.dot_general(q, kk, (((1,), (1,)), ((), ())),
                        preferred_element_type=jnp.float32)
    col = lax.broadcasted_iota(jnp.int32, (64, CHUNK), 1)
    gid = col + c * CHUNK
    s_ref[...] = jnp.where(gid < NKEYS, s, NEG)

    qb = 32
    lane = lax.broadcasted_iota(jnp.int32, (qb, 128), 1) + c * CHUNK
    all_s, all_i = [], []
    for qg in range(64 // qb):
        rs = [jnp.full((qb, 128), NEG, jnp.float32) for _ in range(TOPG)]
        ri = [jnp.full((qb, 128), BIG_ID, jnp.int32) for _ in range(TOPG)]
        for g in range(CHUNK // 128):
            v = s_ref[qg * qb:(qg + 1) * qb, g * 128:(g + 1) * 128]
            vi = lane + (g * 128)
            for t in range(TOPG):
                swap = v > rs[t]
                rs[t], v = jnp.where(swap, v, rs[t]), jnp.where(swap, rs[t], v)
                ri[t], vi = (jnp.where(swap, vi, ri[t]),
                             jnp.where(swap, ri[t], vi))

        cand = jnp.concatenate([r.reshape(qb, 1, 128) for r in rs],
                               axis=1).reshape(qb, TOPG * 128)
        candi = jnp.concatenate([r.reshape(qb, 1, 128) for r in ri],
                                axis=1).reshape(qb, TOPG * 128)
        outs, outis = [], []
        for _ in range(TOPE):
            m = jnp.max(cand, axis=1, keepdims=True)
            idc = jnp.where(cand == m, candi, BIG_ID)
            mi = jnp.min(idc, axis=1, keepdims=True)
            outs.append(m)
            outis.append(mi)
            cand = jnp.where((cand == m) & (candi == mi), NEG, cand)
        all_s.append(jnp.concatenate(outs, axis=1))
        all_i.append(jnp.concatenate(outis, axis=1))
    os_ref[0] = jnp.concatenate(all_s, axis=0)
    oi_ref[0] = jnp.concatenate(all_i, axis=0)


def _select_kernel(cs_ref, ci_ref, ts_ref, ti_ref):
    cand = cs_ref[...]
    candi = ci_ref[...]
    outs, outis = [], []
    for _ in range(K):
        m = jnp.max(cand, axis=1, keepdims=True)
        idc = jnp.where(cand == m, candi, BIG_ID)
        mi = jnp.min(idc, axis=1, keepdims=True)
        outs.append(m)
        outis.append(mi)
        cand = jnp.where((cand == m) & (candi == mi), NEG, cand)
    ts_ref[...] = jnp.concatenate(outs, axis=1)
    ti_ref[...] = jnp.concatenate(outis, axis=1)


def _sc_gather(table1d, idx3):
    nw, r, _ = idx3.shape
    mesh = plsc.VectorSubcoreMesh(core_axis_name="c", subcore_axis_name="s")

    @functools.partial(
        pl.kernel,
        mesh=mesh,
        out_type=jax.ShapeDtypeStruct((nw, r, 128), jnp.float32),
        scratch_types=[
            pltpu.VMEM((r, 128), jnp.int32),
            pltpu.VMEM((r, 128), jnp.float32),
            pltpu.SemaphoreType.DMA,
        ],
    )
    def k(table_hbm, idx_hbm, out_hbm, idx_v, rows_v, sem):
        wid = lax.axis_index("s") * 2 + lax.axis_index("c")
        pltpu.sync_copy(idx_hbm.at[wid], idx_v)
        copies = [
            pltpu.async_copy(table_hbm.at[idx_v.at[j]], rows_v.at[j], sem)
            for j in range(r)
        ]
        for cp in copies:
            cp.wait()
        pltpu.sync_copy(rows_v, out_hbm.at[wid])

    return k(table1d, idx3)


def kernel(queries, keys, k):
    cand_s, cand_i, ktr = pl.pallas_call(
        _chunk_kernel,
        grid=(NCHUNKS,),
        in_specs=[
            pl.BlockSpec((64, 16), lambda c: (0, 0)),
            pl.BlockSpec((CHUNK, 16), lambda c: (c, 0)),
        ],
        out_specs=[
            pl.BlockSpec((1, 64, TOPE), lambda c: (c, 0, 0)),
            pl.BlockSpec((1, 64, TOPE), lambda c: (c, 0, 0)),
            pl.BlockSpec((CHUNK // 128, 16, 128), lambda c: (c, 0, 0)),
        ],
        out_shape=[
            jax.ShapeDtypeStruct((NCHUNKS, 64, TOPE), jnp.float32),
            jax.ShapeDtypeStruct((NCHUNKS, 64, TOPE), jnp.int32),
            jax.ShapeDtypeStruct((NCHUNKS * CHUNK // 128, 16, 128),
                                 jnp.float32),
        ],
        scratch_shapes=[pltpu.VMEM((64, CHUNK), jnp.float32)],
    )(queries, keys)

    cs = cand_s.transpose(1, 0, 2).reshape(64, NCHUNKS * TOPE)
    ci = cand_i.transpose(1, 0, 2).reshape(64, NCHUNKS * TOPE)

    topk_scores, topk_ids = pl.pallas_call(
        _select_kernel,
        out_shape=[
            jax.ShapeDtypeStruct((64, K), jnp.float32),
            jax.ShapeDtypeStruct((64, K), jnp.int32),
        ],
    )(cs, ci)

    base = (topk_ids // 128) * 2048 + (topk_ids % 128)
    idx16 = (base.reshape(64 * K, 1)
             + jnp.arange(16, dtype=jnp.int32)[None, :] * 128
             ).reshape(32, 25, 128)
    emb = _sc_gather(ktr.reshape(NCHUNKS * CHUNK * 16), idx16)
    return (topk_scores, topk_ids, emb.reshape(64, K, 16))
